# baseline (speedup 1.0000x reference)
import jax
import jax.numpy as jnp
from jax.experimental import pallas as pl
from jax.experimental.pallas import tpu as pltpu

_N_PAIRS = 2048
_N_UNARY = 32
_N_BINARY = 16
_BATCH = 4
_GROUP = _N_UNARY + _N_BINARY
_N_GROUPS = _BATCH * _N_PAIRS
_K = 60
_TEMP = 5.0
_TANH_C = 2.5

_NSTREAM = 1
_GQ = _N_GROUPS // _NSTREAM
_GB = 256
_GRID = _GQ // _GB
_RB = _GB * _GROUP


def _head_and_groups(q, H, b3_ref, au_ref, ab_ref, usel_ref, bsel_ref):
    lane = jax.lax.broadcasted_iota(jnp.int32, (1, 128), 1)
    q = jnp.where(lane < _K, q, 0.0)
    A = jax.lax.dot_general(q, H, (((1,), (0,)), ((), ())),
                            preferred_element_type=jnp.float32)
    A3 = A.reshape(_GB, _GROUP, _GROUP)
    ss = jax.lax.broadcasted_iota(jnp.int32, (1, _GROUP, _GROUP), 1)
    ll = jax.lax.broadcasted_iota(jnp.int32, (1, _GROUP, _GROUP), 2)
    o = jnp.sum(jnp.where(ss == ll, A3, 0.0), axis=1) + b3_ref[...]
    o = jnp.where(o > 0, o, jnp.exp(jnp.minimum(o, 0.0)) - 1.0)
    l = _TANH_C * jnp.tanh(o * (1.0 / _TEMP))
    e = jnp.exp(l)
    k = jax.lax.broadcasted_iota(jnp.int32, e.shape, 1)
    is_u = k < _N_UNARY
    su = jnp.sum(jnp.where(is_u, e, 0.0), axis=1, keepdims=True)
    sb = jnp.sum(jnp.where(is_u, 0.0, e), axis=1, keepdims=True)
    sel_u = jnp.sum(jnp.where(k == au_ref[...], e, 0.0), axis=1, keepdims=True)
    sel_b = jnp.sum(jnp.where(k == ab_ref[...] + _N_UNARY, e, 0.0),
                    axis=1, keepdims=True)
    usel_ref[...] = sel_u / su
    bsel_ref[...] = sel_b / sb


def _fused_kernel(x_ref, W1_ref, b1_ref, W2_ref, b2_ref, *refs):
    w_refs = refs[:_NSTREAM]
    b3_refs = refs[_NSTREAM:2 * _NSTREAM]
    au_refs = refs[2 * _NSTREAM:3 * _NSTREAM]
    ab_refs = refs[3 * _NSTREAM:4 * _NSTREAM]
    usel_refs = refs[4 * _NSTREAM:5 * _NSTREAM]
    bsel_refs = refs[5 * _NSTREAM:6 * _NSTREAM]
    H_ref = refs[6 * _NSTREAM]

    @pl.when(pl.program_id(0) == 0)
    def _build_head():
        h = jnp.tanh(
            jax.lax.dot_general(x_ref[...], W1_ref[...],
                                (((1,), (1,)), ((), ())),
                                preferred_element_type=jnp.float32)
            + b1_ref[...])
        h2 = jnp.tanh(
            jax.lax.dot_general(h, W2_ref[...],
                                (((1,), (1,)), ((), ())),
                                preferred_element_type=jnp.float32)
            + b2_ref[...])
        ii = jax.lax.broadcasted_iota(jnp.int32, (128, _K), 0)
        jj = jax.lax.broadcasted_iota(jnp.int32, (128, _K), 1)
        eye = (ii == jj).astype(jnp.float32)
        h2col = jax.lax.dot_general(eye, h2, (((1,), (1,)), ((), ())),
                                    preferred_element_type=jnp.float32)
        H_ref[...] = jnp.broadcast_to(h2col, (128, _GROUP))

    H = H_ref[...]
    for s in range(_NSTREAM):
        _head_and_groups(w_refs[s][...], H, b3_refs[s], au_refs[s],
                         ab_refs[s], usel_refs[s], bsel_refs[s])


def kernel(x, W1, b1, W2, b2, W3, b3):
    skey = jax.random.key(42)
    ku, kb = jax.random.split(skey)
    u_act = jax.random.randint(ku, (_BATCH, _N_PAIRS), 0, _N_UNARY)
    b_act = jax.random.randint(kb, (_BATCH, _N_PAIRS), 0, _N_BINARY)

    b3g = b3.reshape(_N_GROUPS, _GROUP)
    au = u_act.reshape(_N_GROUPS, 1).astype(jnp.int32)
    ab = b_act.reshape(_N_GROUPS, 1).astype(jnp.int32)

    full = lambda shp: pl.BlockSpec(shp, lambda i: (0, 0))

    def w_spec(s):
        return pl.BlockSpec((_RB, 128), lambda i, s=s: (s * _GRID + i, 0))

    def g_spec(s, width):
        return pl.BlockSpec((_GB, width), lambda i, s=s: (s * _GRID + i, 0))

    in_specs = ([full((1, 20)), full((60, 20)), full((1, 60)),
                 full((60, 60)), full((1, 60))]
                + [w_spec(s) for s in range(_NSTREAM)]
                + [g_spec(s, _GROUP) for s in range(_NSTREAM)]
                + [g_spec(s, 1) for s in range(_NSTREAM)]
                + [g_spec(s, 1) for s in range(_NSTREAM)])
    out_specs = [pl.BlockSpec((_GB, 1), lambda i: (i, 0))
                 for _ in range(2 * _NSTREAM)]
    out_shape = [jax.ShapeDtypeStruct((_GQ, 1), jnp.float32)
                 for _ in range(2 * _NSTREAM)]

    outs = pl.pallas_call(
        _fused_kernel,
        grid=(_GRID,),
        in_specs=in_specs,
        out_specs=out_specs,
        out_shape=out_shape,
        scratch_shapes=[pltpu.VMEM((128, _GROUP), jnp.float32)],
    )(x, W1, b1.reshape(1, 60), W2, b2.reshape(1, 60),
      *([W3] * _NSTREAM),
      *[b3g] * _NSTREAM, *[au] * _NSTREAM, *[ab] * _NSTREAM)

    usel = jnp.concatenate(outs[:_NSTREAM], axis=0)
    bsel = jnp.concatenate(outs[_NSTREAM:], axis=0)

    actions = jnp.stack([u_act, b_act], axis=-1).reshape(
        _BATCH, 2 * _N_PAIRS).astype(jnp.int32)
    sel_probs = jnp.stack(
        [usel.reshape(_BATCH, _N_PAIRS), bsel.reshape(_BATCH, _N_PAIRS)],
        axis=-1).reshape(_BATCH, 2 * _N_PAIRS)
    return actions, sel_probs


# speedup vs baseline: 1.4083x; 1.4083x over previous
import jax
import jax.numpy as jnp
from jax.experimental import pallas as pl
from jax.experimental.pallas import tpu as pltpu

_N_PAIRS = 2048
_N_UNARY = 32
_N_BINARY = 16
_BATCH = 4
_GROUP = _N_UNARY + _N_BINARY
_N_GROUPS = _BATCH * _N_PAIRS
_K = 60
_TEMP = 5.0
_TANH_C = 2.5

_NSTREAM = 1
_GQ = _N_GROUPS // _NSTREAM
_GB = 256
_GRID = _GQ // _GB
_RB = _GB * _GROUP


def _head_and_groups(q, H, b3_ref, au_ref, ab_ref, usel_ref, bsel_ref):
    q = q.reshape(_RB, _K)
    A = jax.lax.dot_general(q, H, (((1,), (0,)), ((), ())),
                            preferred_element_type=jnp.float32)
    A3 = A.reshape(_GB, _GROUP, _GROUP)
    ss = jax.lax.broadcasted_iota(jnp.int32, (1, _GROUP, _GROUP), 1)
    ll = jax.lax.broadcasted_iota(jnp.int32, (1, _GROUP, _GROUP), 2)
    o = jnp.sum(jnp.where(ss == ll, A3, 0.0), axis=1) + b3_ref[...]
    o = jnp.where(o > 0, o, jnp.exp(jnp.minimum(o, 0.0)) - 1.0)
    l = _TANH_C * jnp.tanh(o * (1.0 / _TEMP))
    e = jnp.exp(l)
    k = jax.lax.broadcasted_iota(jnp.int32, e.shape, 1)
    is_u = k < _N_UNARY
    su = jnp.sum(jnp.where(is_u, e, 0.0), axis=1, keepdims=True)
    sb = jnp.sum(jnp.where(is_u, 0.0, e), axis=1, keepdims=True)
    sel_u = jnp.sum(jnp.where(k == au_ref[...], e, 0.0), axis=1, keepdims=True)
    sel_b = jnp.sum(jnp.where(k == ab_ref[...] + _N_UNARY, e, 0.0),
                    axis=1, keepdims=True)
    usel_ref[...] = sel_u / su
    bsel_ref[...] = sel_b / sb


def _fused_kernel(x_ref, W1_ref, b1_ref, W2_ref, b2_ref, *refs):
    w_refs = refs[:_NSTREAM]
    b3_refs = refs[_NSTREAM:2 * _NSTREAM]
    au_refs = refs[2 * _NSTREAM:3 * _NSTREAM]
    ab_refs = refs[3 * _NSTREAM:4 * _NSTREAM]
    usel_refs = refs[4 * _NSTREAM:5 * _NSTREAM]
    bsel_refs = refs[5 * _NSTREAM:6 * _NSTREAM]
    H_ref = refs[6 * _NSTREAM]

    @pl.when(pl.program_id(0) == 0)
    def _build_head():
        h = jnp.tanh(
            jax.lax.dot_general(x_ref[...], W1_ref[...],
                                (((1,), (1,)), ((), ())),
                                preferred_element_type=jnp.float32)
            + b1_ref[...])
        h2 = jnp.tanh(
            jax.lax.dot_general(h, W2_ref[...],
                                (((1,), (1,)), ((), ())),
                                preferred_element_type=jnp.float32)
            + b2_ref[...])
        ii = jax.lax.broadcasted_iota(jnp.int32, (_K, _K), 0)
        jj = jax.lax.broadcasted_iota(jnp.int32, (_K, _K), 1)
        eye = (ii == jj).astype(jnp.float32)
        h2col = jax.lax.dot_general(eye, h2, (((1,), (1,)), ((), ())),
                                    preferred_element_type=jnp.float32)
        H_ref[...] = jnp.broadcast_to(h2col, (_K, _GROUP))

    H = H_ref[...]
    for s in range(_NSTREAM):
        _head_and_groups(w_refs[s][...], H, b3_refs[s], au_refs[s],
                         ab_refs[s], usel_refs[s], bsel_refs[s])


def kernel(x, W1, b1, W2, b2, W3, b3):
    skey = jax.random.key(42)
    ku, kb = jax.random.split(skey)
    u_act = jax.random.randint(ku, (_BATCH, _N_PAIRS), 0, _N_UNARY)
    b_act = jax.random.randint(kb, (_BATCH, _N_PAIRS), 0, _N_BINARY)

    b3g = b3.reshape(_N_GROUPS, _GROUP)
    au = u_act.reshape(_N_GROUPS, 1).astype(jnp.int32)
    ab = b_act.reshape(_N_GROUPS, 1).astype(jnp.int32)

    full = lambda shp: pl.BlockSpec(shp, lambda i: (0, 0))

    def w_spec(s):
        return pl.BlockSpec((_RB // 8, 8, _K),
                            lambda i, s=s: (s * _GRID + i, 0, 0))

    def g_spec(s, width):
        return pl.BlockSpec((_GB, width), lambda i, s=s: (s * _GRID + i, 0))

    in_specs = ([full((1, 20)), full((60, 20)), full((1, 60)),
                 full((60, 60)), full((1, 60))]
                + [w_spec(s) for s in range(_NSTREAM)]
                + [g_spec(s, _GROUP) for s in range(_NSTREAM)]
                + [g_spec(s, 1) for s in range(_NSTREAM)]
                + [g_spec(s, 1) for s in range(_NSTREAM)])
    out_specs = [pl.BlockSpec((_GB, 1), lambda i: (i, 0))
                 for _ in range(2 * _NSTREAM)]
    out_shape = [jax.ShapeDtypeStruct((_GQ, 1), jnp.float32)
                 for _ in range(2 * _NSTREAM)]

    outs = pl.pallas_call(
        _fused_kernel,
        grid=(_GRID,),
        in_specs=in_specs,
        out_specs=out_specs,
        out_shape=out_shape,
        scratch_shapes=[pltpu.VMEM((_K, _GROUP), jnp.float32)],
    )(x, W1, b1.reshape(1, 60), W2, b2.reshape(1, 60),
      *([W3.reshape(_N_GROUPS * _GROUP // 8, 8, _K)] * _NSTREAM),
      *[b3g] * _NSTREAM, *[au] * _NSTREAM, *[ab] * _NSTREAM)

    usel = jnp.concatenate(outs[:_NSTREAM], axis=0)
    bsel = jnp.concatenate(outs[_NSTREAM:], axis=0)

    actions = jnp.stack([u_act, b_act], axis=-1).reshape(
        _BATCH, 2 * _N_PAIRS).astype(jnp.int32)
    sel_probs = jnp.stack(
        [usel.reshape(_BATCH, _N_PAIRS), bsel.reshape(_BATCH, _N_PAIRS)],
        axis=-1).reshape(_BATCH, 2 * _N_PAIRS)
    return actions, sel_probs
